# Initial kernel scaffold; baseline (speedup 1.0000x reference)
import functools

import jax
import jax.numpy as jnp
from jax import lax
from jax.experimental import pallas as pl
from jax.experimental.pallas import tpu as pltpu
from jax.experimental.pallas import tpu_sc as plsc

_VOCAB = 100000
_EMBED = 128
_OUT = 64
_BATCH = 4096
_SEQ = 200
_LANES = 16

_CHUNK = 104
_NCHUNK = 2
_SEQ_PAD = _CHUNK * _NCHUNK

_PROJ_BLK = 2000

_info = plsc.get_sparse_core_info()
_NC, _NS = _info.num_cores, _info.num_subcores
_NW = _NC * _NS
_BPW = _BATCH // _NW


def _proj_body(emb_ref, w_ref, out_ref):
    blk = lax.dot_general(
        emb_ref[...], w_ref[...],
        dimension_numbers=(((1,), (1,)), ((), ())),
        preferred_element_type=jnp.float32,
    ) * (1.0 / _SEQ)
    row = (lax.broadcasted_iota(jnp.int32, blk.shape, 0)
           + pl.program_id(0) * _PROJ_BLK)
    out_ref[...] = jnp.where(row == 0, 0.0, blk)


def _project(emb, w):
    return pl.pallas_call(
        _proj_body,
        grid=(_VOCAB // _PROJ_BLK,),
        in_specs=[
            pl.BlockSpec((_PROJ_BLK, _EMBED), lambda i: (i, 0)),
            pl.BlockSpec((_OUT, _EMBED), lambda i: (0, 0)),
        ],
        out_specs=pl.BlockSpec((_PROJ_BLK, _OUT), lambda i: (i, 0)),
        out_shape=jax.ShapeDtypeStruct((_VOCAB, _OUT), jnp.float32),
    )(emb, w)


@functools.partial(
    pl.kernel,
    out_type=jax.ShapeDtypeStruct((_BATCH, _OUT), jnp.float32),
    mesh=plsc.VectorSubcoreMesh(core_axis_name="c", subcore_axis_name="s"),
    scratch_types=[
        pltpu.VMEM((_BPW, _SEQ_PAD), jnp.int32),
        pltpu.VMEM((2, _SEQ_PAD, _OUT), jnp.float32),
        pltpu.VMEM((_BPW, _OUT), jnp.float32),
        pltpu.VMEM((_OUT,), jnp.float32),
        pltpu.SemaphoreType.DMA,
        pltpu.SemaphoreType.DMA,
    ],
)
def _pool(idx_hbm, p_hbm, b_hbm, out_hbm, idx_v, rows_v, out_v, bias_v,
          sem0, sem1):
    wid = lax.axis_index("s") * _NC + lax.axis_index("c")
    base = wid * _BPW
    pltpu.sync_copy(idx_hbm.at[pl.ds(base, _BPW)], idx_v)
    pltpu.sync_copy(b_hbm, bias_v)
    sems = (sem0, sem1)

    def _start(i, buf):
        for j in range(_NCHUNK):
            pltpu.make_async_copy(
                p_hbm.at[idx_v.at[i, pl.ds(j * _CHUNK, _CHUNK)]],
                rows_v.at[buf, pl.ds(j * _CHUNK, _CHUNK)],
                sems[buf],
            ).start()

    def _wait(buf):
        for j in range(_NCHUNK):
            pltpu.make_async_copy(
                p_hbm.at[idx_v.at[0, pl.ds(j * _CHUNK, _CHUNK)]],
                rows_v.at[buf, pl.ds(j * _CHUNK, _CHUNK)],
                sems[buf],
            ).wait()

    def _accum_row(buf, i):
        def s_body(s, accs):
            return tuple(accs[k] + rows_v[buf, s, pl.ds(_LANES * k, _LANES)]
                         for k in range(_OUT // _LANES))

        accs = lax.fori_loop(
            0, _SEQ_PAD, s_body,
            tuple(bias_v[pl.ds(_LANES * k, _LANES)]
                  for k in range(_OUT // _LANES)),
            unroll=4,
        )
        for k in range(_OUT // _LANES):
            out_v[i, pl.ds(_LANES * k, _LANES)] = accs[k]

    _start(0, 0)

    def pair_body(ii, carry):
        i0 = ii * 2
        _start(i0 + 1, 1)
        _wait(0)
        _accum_row(0, i0)

        @pl.when(i0 + 2 < _BPW)
        def _():
            _start(i0 + 2, 0)

        _wait(1)
        _accum_row(1, i0 + 1)
        return carry

    lax.fori_loop(0, _BPW // 2, pair_body, 0)
    pltpu.sync_copy(out_v, out_hbm.at[pl.ds(base, _BPW)])


def kernel(x, emb, W, b):
    idx = jnp.pad(x.astype(jnp.int32), ((0, 0), (0, _SEQ_PAD - _SEQ)))
    p = _project(emb, W)
    return _pool(idx, p, b)


# speedup vs baseline: 3.4499x; 3.4499x over previous
import functools

import jax
import jax.numpy as jnp
from jax import lax
from jax.experimental import pallas as pl
from jax.experimental.pallas import tpu as pltpu
from jax.experimental.pallas import tpu_sc as plsc

_VOCAB = 100000
_EMBED = 128
_OUT = 64
_BATCH = 4096
_SEQ = 200
_LANES = 16

_CHUNK = 104
_NCHUNK = 2
_SEQ_PAD = _CHUNK * _NCHUNK

_PROJ_BLK = 2000

_info = plsc.get_sparse_core_info()
_NC, _NS = _info.num_cores, _info.num_subcores
_NW = _NC * _NS
_BPW = _BATCH // _NW


def _proj_body(emb_ref, w_ref, out_ref):
    blk = lax.dot_general(
        emb_ref[...], w_ref[...],
        dimension_numbers=(((1,), (1,)), ((), ())),
        preferred_element_type=jnp.float32,
    ) * (1.0 / _SEQ)
    row = (lax.broadcasted_iota(jnp.int32, blk.shape, 0)
           + pl.program_id(0) * _PROJ_BLK)
    out_ref[...] = jnp.where(row == 0, 0.0, blk)


def _project(emb, w):
    return pl.pallas_call(
        _proj_body,
        grid=(_VOCAB // _PROJ_BLK,),
        in_specs=[
            pl.BlockSpec((_PROJ_BLK, _EMBED), lambda i: (i, 0)),
            pl.BlockSpec((_OUT, _EMBED), lambda i: (0, 0)),
        ],
        out_specs=pl.BlockSpec((_PROJ_BLK, _OUT), lambda i: (i, 0)),
        out_shape=jax.ShapeDtypeStruct((_VOCAB, _OUT), jnp.float32),
    )(emb, w)


@functools.partial(
    pl.kernel,
    out_type=jax.ShapeDtypeStruct((_BATCH, _OUT), jnp.float32),
    mesh=plsc.VectorSubcoreMesh(core_axis_name="c", subcore_axis_name="s"),
    compiler_params=pltpu.CompilerParams(use_tc_tiling_on_sc=False),
    scratch_types=[
        pltpu.VMEM((_BPW * _SEQ_PAD,), jnp.int32),
        pltpu.VMEM((2, _SEQ_PAD, _OUT), jnp.float32),
        pltpu.VMEM((_BPW, _OUT), jnp.float32),
        pltpu.VMEM((_OUT,), jnp.float32),
        pltpu.SemaphoreType.DMA,
        pltpu.SemaphoreType.DMA,
    ],
)
def _pool(idx_hbm, p_hbm, b_hbm, out_hbm, idx_v, rows_v, out_v, bias_v,
          sem0, sem1):
    wid = lax.axis_index("s") * _NC + lax.axis_index("c")
    base = wid * _BPW
    pltpu.sync_copy(idx_hbm.at[pl.ds(base * _SEQ_PAD, _BPW * _SEQ_PAD)], idx_v)
    pltpu.sync_copy(b_hbm, bias_v)
    sems = (sem0, sem1)

    def _start(i, buf):
        for j in range(_NCHUNK):
            off = pl.multiple_of(i * _SEQ_PAD + j * _CHUNK, 8)
            pltpu.make_async_copy(
                p_hbm.at[idx_v.at[pl.ds(off, _CHUNK)]],
                rows_v.at[buf, pl.ds(j * _CHUNK, _CHUNK)],
                sems[buf],
            ).start()

    def _wait(buf):
        for j in range(_NCHUNK):
            pltpu.make_async_copy(
                p_hbm.at[idx_v.at[pl.ds(j * _CHUNK, _CHUNK)]],
                rows_v.at[buf, pl.ds(j * _CHUNK, _CHUNK)],
                sems[buf],
            ).wait()

    def _accum_row(buf, i):
        def s_body(s, accs):
            return tuple(accs[k] + rows_v[buf, s, pl.ds(_LANES * k, _LANES)]
                         for k in range(_OUT // _LANES))

        accs = lax.fori_loop(
            0, _SEQ_PAD, s_body,
            tuple(bias_v[pl.ds(_LANES * k, _LANES)]
                  for k in range(_OUT // _LANES)),
            unroll=4,
        )
        for k in range(_OUT // _LANES):
            out_v[i, pl.ds(_LANES * k, _LANES)] = accs[k]

    _start(0, 0)

    def pair_body(ii, carry):
        i0 = ii * 2
        _start(i0 + 1, 1)
        _wait(0)
        _accum_row(0, i0)

        @pl.when(i0 + 2 < _BPW)
        def _():
            _start(i0 + 2, 0)

        _wait(1)
        _accum_row(1, i0 + 1)
        return carry

    lax.fori_loop(0, _BPW // 2, pair_body, 0)
    pltpu.sync_copy(out_v, out_hbm.at[pl.ds(base, _BPW)])


def kernel(x, emb, W, b):
    idx = jnp.pad(x.astype(jnp.int32), ((0, 0), (0, _SEQ_PAD - _SEQ)))
    p = _project(emb, W)
    return _pool(idx.reshape(-1), p, b)
